# Initial kernel scaffold; baseline (speedup 1.0000x reference)
import functools
import math

import jax
import jax.numpy as jnp
from jax import lax
from jax.experimental import pallas as pl
from jax.experimental.pallas import tpu as pltpu
from jax.experimental.pallas import tpu_sc as plsc

N = 10000
E = 320000
D = 128

NC = 2
NS = 16
NW = NC * NS
EPW = E // NW
CH = 80
NCHUNK = EPW // CH
RPT = N // NS
DEGW = 16

_SC_MESH = dict(core_axis_name="c", subcore_axis_name="s",
                num_cores=NC, num_subcores=NS)


def _sc_degree(dst, deg_init):
    mesh = plsc.VectorSubcoreMesh(**_SC_MESH)

    @functools.partial(
        pl.kernel,
        out_type=jax.ShapeDtypeStruct((NC, N, DEGW), jnp.float32),
        mesh=mesh,
        scratch_types=[
            pltpu.VMEM((1, CH), jnp.int32),
            pltpu.VMEM((CH, DEGW), jnp.float32),
            pltpu.VMEM_SHARED((N, DEGW), jnp.float32),
        ],
    )
    def deg_kernel(dst_hbm, init_hbm, out_hbm, idx_v, ones_v, acc_sh):
        c = lax.axis_index("c")
        s = lax.axis_index("s")
        wid = s * NC + c

        def fill(r, carry):
            ones_v[r, :] = jnp.full((DEGW,), 1.0, jnp.float32)
            return carry
        lax.fori_loop(0, CH, fill, 0)

        pltpu.sync_copy(init_hbm.at[c, pl.ds(s * RPT, RPT)],
                        acc_sh.at[pl.ds(s * RPT, RPT)])
        plsc.subcore_barrier()

        def body(i, carry):
            base = wid * EPW + i * CH
            pltpu.sync_copy(dst_hbm.at[pl.ds(base, CH)], idx_v.at[0])
            pltpu.sync_copy(ones_v, acc_sh.at[idx_v.at[0]], add=True)
            return carry
        lax.fori_loop(0, NCHUNK, body, 0)

        plsc.subcore_barrier()
        pltpu.sync_copy(acc_sh.at[pl.ds(s * RPT, RPT)],
                        out_hbm.at[c, pl.ds(s * RPT, RPT)])

    return deg_kernel(dst, deg_init)


def _sc_scatter(g, src, dst, zeros):
    mesh = plsc.VectorSubcoreMesh(**_SC_MESH)

    @functools.partial(
        pl.kernel,
        out_type=jax.ShapeDtypeStruct((NC, N, D), jnp.float32),
        mesh=mesh,
        scratch_types=[
            pltpu.VMEM((1, CH), jnp.int32),
            pltpu.VMEM((1, CH), jnp.int32),
            pltpu.VMEM((CH, D), jnp.float32),
            pltpu.SemaphoreType.DMA,
            pltpu.VMEM_SHARED((N, D), jnp.float32),
        ],
    )
    def scatter_kernel(g_hbm, src_hbm, dst_hbm, z_hbm, out_hbm,
                       si_v, di_v, rows_v, sem, acc_sh):
        c = lax.axis_index("c")
        s = lax.axis_index("s")
        wid = s * NC + c

        @pl.when(c == 0)
        def _():
            pltpu.sync_copy(g_hbm.at[pl.ds(s * RPT, RPT)],
                            acc_sh.at[pl.ds(s * RPT, RPT)])

        @pl.when(c != 0)
        def _():
            pltpu.sync_copy(z_hbm.at[pl.ds(s * RPT, RPT)],
                            acc_sh.at[pl.ds(s * RPT, RPT)])

        plsc.subcore_barrier()

        def body(i, carry):
            base = wid * EPW + i * CH
            pltpu.sync_copy(src_hbm.at[pl.ds(base, CH)], si_v.at[0])
            pltpu.sync_copy(dst_hbm.at[pl.ds(base, CH)], di_v.at[0])
            pltpu.async_copy(g_hbm.at[si_v.at[0]], rows_v, sem).wait()
            pltpu.sync_copy(rows_v, acc_sh.at[di_v.at[0]], add=True)
            return carry
        lax.fori_loop(0, NCHUNK, body, 0)

        plsc.subcore_barrier()
        pltpu.sync_copy(acc_sh.at[pl.ds(s * RPT, RPT)],
                        out_hbm.at[c, pl.ds(s * RPT, RPT)])

    return scatter_kernel(g, src, dst, zeros)


_RB = 1000
_GRID = N // _RB


def _tc1_body(x_ref, w_ref, deg_ref, g_ref, dinv_ref):
    h = jnp.dot(x_ref[...], w_ref[...], preferred_element_type=jnp.float32)
    deg = deg_ref[0, :, 0:1] + deg_ref[1, :, 0:1]
    dinv = lax.rsqrt(deg)
    g_ref[...] = h * dinv
    dinv_ref[...] = jnp.broadcast_to(dinv, (_RB, DEGW))


def _tc1(x, W1, deg2):
    return pl.pallas_call(
        _tc1_body,
        grid=(_GRID,),
        in_specs=[
            pl.BlockSpec((_RB, D), lambda r: (r, 0)),
            pl.BlockSpec((D, D), lambda r: (0, 0)),
            pl.BlockSpec((NC, _RB, DEGW), lambda r: (0, r, 0)),
        ],
        out_specs=[
            pl.BlockSpec((_RB, D), lambda r: (r, 0)),
            pl.BlockSpec((_RB, DEGW), lambda r: (r, 0)),
        ],
        out_shape=[
            jax.ShapeDtypeStruct((N, D), jnp.float32),
            jax.ShapeDtypeStruct((N, DEGW), jnp.float32),
        ],
    )(x, W1, deg2)


def _tc2_body(agg_ref, dinv_ref, b1_ref, w2_ref, g2_ref):
    a = agg_ref[0] + agg_ref[1]
    dinv = dinv_ref[:, 0:1]
    z = a * dinv + b1_ref[...]
    h = z * 0.5 * (1.0 + lax.erf(z * (1.0 / math.sqrt(2.0))))
    h2 = jnp.dot(h, w2_ref[...], preferred_element_type=jnp.float32)
    g2_ref[...] = h2 * dinv


def _tc2(agg1, dinv, b1, W2):
    return pl.pallas_call(
        _tc2_body,
        grid=(_GRID,),
        in_specs=[
            pl.BlockSpec((NC, _RB, D), lambda r: (0, r, 0)),
            pl.BlockSpec((_RB, DEGW), lambda r: (r, 0)),
            pl.BlockSpec((1, D), lambda r: (0, 0)),
            pl.BlockSpec((D, D), lambda r: (0, 0)),
        ],
        out_specs=pl.BlockSpec((_RB, D), lambda r: (r, 0)),
        out_shape=jax.ShapeDtypeStruct((N, D), jnp.float32),
    )(agg1, dinv, b1, W2)


def _tc3_body(agg_ref, dinv_ref, b2_ref, out_ref):
    a = agg_ref[0] + agg_ref[1]
    out_ref[...] = a * dinv_ref[:, 0:1] + b2_ref[...]


def _tc3(agg2, dinv, b2):
    return pl.pallas_call(
        _tc3_body,
        grid=(_GRID,),
        in_specs=[
            pl.BlockSpec((NC, _RB, D), lambda r: (0, r, 0)),
            pl.BlockSpec((_RB, DEGW), lambda r: (r, 0)),
            pl.BlockSpec((1, D), lambda r: (0, 0)),
        ],
        out_specs=pl.BlockSpec((_RB, D), lambda r: (r, 0)),
        out_shape=jax.ShapeDtypeStruct((N, D), jnp.float32),
    )(agg2, dinv, b2)


@jax.jit
def _run(x, edge_index, target, W1, b1, W2, b2):
    src = edge_index[0]
    dst = edge_index[1]
    zeros = jnp.zeros((N, D), jnp.float32)
    deg_init = jnp.stack([jnp.ones((N, DEGW), jnp.float32),
                          jnp.zeros((N, DEGW), jnp.float32)])
    b1r = b1.reshape(1, D)
    b2r = b2.reshape(1, D)

    deg2 = _sc_degree(dst, deg_init)
    g1, dinv = _tc1(x, W1, deg2)
    agg1 = _sc_scatter(g1, src, dst, zeros)
    g2 = _tc2(agg1, dinv, b1r, W2)
    agg2 = _sc_scatter(g2, src, dst, zeros)
    out = _tc3(agg2, dinv, b2r)
    return (out, target)


def kernel(x, edge_index, target, W1, b1, W2, b2):
    return _run(x, edge_index, target, W1, b1, W2, b2)


# speedup vs baseline: 12.8652x; 12.8652x over previous
import functools
import math

import jax
import jax.numpy as jnp
from jax import lax
from jax.experimental import pallas as pl
from jax.experimental.pallas import tpu as pltpu
from jax.experimental.pallas import tpu_sc as plsc

N = 10000
E = 320000
D = 128

NC = 2
NS = 16
NW = NC * NS
EPW = E // NW
CH = 80
NCHUNK = EPW // CH
RPT = 624
RPT_LAST = N - RPT * (NS - 1)
DEGW = 16

_SC_MESH = dict(core_axis_name="c", subcore_axis_name="s",
                num_cores=NC, num_subcores=NS)


def _rows_partition(s, fn):
    @pl.when(s < NS - 1)
    def _():
        fn(pl.multiple_of(s * RPT, 8), RPT)

    @pl.when(s == NS - 1)
    def _():
        fn((NS - 1) * RPT, RPT_LAST)


def _sc_degree(dst, deg_init):
    mesh = plsc.VectorSubcoreMesh(**_SC_MESH)

    @functools.partial(
        pl.kernel,
        out_type=jax.ShapeDtypeStruct((NC, N, DEGW), jnp.float32),
        mesh=mesh,
        scratch_types=[
            pltpu.VMEM((1, CH), jnp.int32),
            pltpu.VMEM((CH, DEGW), jnp.float32),
            pltpu.VMEM_SHARED((N, DEGW), jnp.float32),
        ],
    )
    def deg_kernel(dst_hbm, init_hbm, out_hbm, idx_v, ones_v, acc_sh):
        c = lax.axis_index("c")
        s = lax.axis_index("s")
        wid = s * NC + c

        def fill(r, carry):
            ones_v[r, :] = jnp.full((DEGW,), 1.0, jnp.float32)
            return carry
        lax.fori_loop(0, CH, fill, 0)

        _rows_partition(s, lambda r0, n: pltpu.sync_copy(
            init_hbm.at[c, pl.ds(r0, n)], acc_sh.at[pl.ds(r0, n)]))
        plsc.subcore_barrier()

        def body(i, carry):
            base = wid * EPW + i * CH
            pltpu.sync_copy(dst_hbm.at[pl.ds(base, CH)], idx_v.at[0])
            pltpu.sync_copy(ones_v, acc_sh.at[idx_v.at[0]], add=True)
            return carry
        lax.fori_loop(0, NCHUNK, body, 0)

        plsc.subcore_barrier()
        _rows_partition(s, lambda r0, n: pltpu.sync_copy(
            acc_sh.at[pl.ds(r0, n)], out_hbm.at[c, pl.ds(r0, n)]))

    return deg_kernel(dst, deg_init)


def _sc_scatter(g, src, dst, zeros):
    mesh = plsc.VectorSubcoreMesh(**_SC_MESH)

    @functools.partial(
        pl.kernel,
        out_type=jax.ShapeDtypeStruct((NC, N, D), jnp.float32),
        mesh=mesh,
        scratch_types=[
            pltpu.VMEM((1, CH), jnp.int32),
            pltpu.VMEM((1, CH), jnp.int32),
            pltpu.VMEM((CH, D), jnp.float32),
            pltpu.SemaphoreType.DMA,
            pltpu.VMEM_SHARED((N, D), jnp.float32),
        ],
    )
    def scatter_kernel(g_hbm, src_hbm, dst_hbm, z_hbm, out_hbm,
                       si_v, di_v, rows_v, sem, acc_sh):
        c = lax.axis_index("c")
        s = lax.axis_index("s")
        wid = s * NC + c

        @pl.when(c == 0)
        def _():
            _rows_partition(s, lambda r0, n: pltpu.sync_copy(
                g_hbm.at[pl.ds(r0, n)], acc_sh.at[pl.ds(r0, n)]))

        @pl.when(c != 0)
        def _():
            _rows_partition(s, lambda r0, n: pltpu.sync_copy(
                z_hbm.at[pl.ds(r0, n)], acc_sh.at[pl.ds(r0, n)]))

        plsc.subcore_barrier()

        def body(i, carry):
            base = wid * EPW + i * CH
            pltpu.sync_copy(src_hbm.at[pl.ds(base, CH)], si_v.at[0])
            pltpu.sync_copy(dst_hbm.at[pl.ds(base, CH)], di_v.at[0])
            pltpu.async_copy(g_hbm.at[si_v.at[0]], rows_v, sem).wait()
            pltpu.sync_copy(rows_v, acc_sh.at[di_v.at[0]], add=True)
            return carry
        lax.fori_loop(0, NCHUNK, body, 0)

        plsc.subcore_barrier()
        _rows_partition(s, lambda r0, n: pltpu.sync_copy(
            acc_sh.at[pl.ds(r0, n)], out_hbm.at[c, pl.ds(r0, n)]))

    return scatter_kernel(g, src, dst, zeros)


_RB = 1000
_GRID = N // _RB


def _tc1_body(x_ref, w_ref, deg_ref, g_ref, dinv_ref):
    h = jnp.dot(x_ref[...], w_ref[...], preferred_element_type=jnp.float32)
    deg = deg_ref[0, :, 0:1] + deg_ref[1, :, 0:1]
    dinv = lax.rsqrt(deg)
    g_ref[...] = h * dinv
    dinv_ref[...] = jnp.broadcast_to(dinv, (_RB, DEGW))


def _tc1(x, W1, deg2):
    return pl.pallas_call(
        _tc1_body,
        grid=(_GRID,),
        in_specs=[
            pl.BlockSpec((_RB, D), lambda r: (r, 0)),
            pl.BlockSpec((D, D), lambda r: (0, 0)),
            pl.BlockSpec((NC, _RB, DEGW), lambda r: (0, r, 0)),
        ],
        out_specs=[
            pl.BlockSpec((_RB, D), lambda r: (r, 0)),
            pl.BlockSpec((_RB, DEGW), lambda r: (r, 0)),
        ],
        out_shape=[
            jax.ShapeDtypeStruct((N, D), jnp.float32),
            jax.ShapeDtypeStruct((N, DEGW), jnp.float32),
        ],
    )(x, W1, deg2)


def _tc2_body(agg_ref, dinv_ref, b1_ref, w2_ref, g2_ref):
    a = agg_ref[0] + agg_ref[1]
    dinv = dinv_ref[:, 0:1]
    z = a * dinv + b1_ref[...]
    h = z * 0.5 * (1.0 + lax.erf(z * (1.0 / math.sqrt(2.0))))
    h2 = jnp.dot(h, w2_ref[...], preferred_element_type=jnp.float32)
    g2_ref[...] = h2 * dinv


def _tc2(agg1, dinv, b1, W2):
    return pl.pallas_call(
        _tc2_body,
        grid=(_GRID,),
        in_specs=[
            pl.BlockSpec((NC, _RB, D), lambda r: (0, r, 0)),
            pl.BlockSpec((_RB, DEGW), lambda r: (r, 0)),
            pl.BlockSpec((1, D), lambda r: (0, 0)),
            pl.BlockSpec((D, D), lambda r: (0, 0)),
        ],
        out_specs=pl.BlockSpec((_RB, D), lambda r: (r, 0)),
        out_shape=jax.ShapeDtypeStruct((N, D), jnp.float32),
    )(agg1, dinv, b1, W2)


def _tc3_body(agg_ref, dinv_ref, b2_ref, out_ref):
    a = agg_ref[0] + agg_ref[1]
    out_ref[...] = a * dinv_ref[:, 0:1] + b2_ref[...]


def _tc3(agg2, dinv, b2):
    return pl.pallas_call(
        _tc3_body,
        grid=(_GRID,),
        in_specs=[
            pl.BlockSpec((NC, _RB, D), lambda r: (0, r, 0)),
            pl.BlockSpec((_RB, DEGW), lambda r: (r, 0)),
            pl.BlockSpec((1, D), lambda r: (0, 0)),
        ],
        out_specs=pl.BlockSpec((_RB, D), lambda r: (r, 0)),
        out_shape=jax.ShapeDtypeStruct((N, D), jnp.float32),
    )(agg2, dinv, b2)


@jax.jit
def _run(x, edge_index, target, W1, b1, W2, b2):
    src = edge_index[0]
    dst = edge_index[1]
    zeros = jnp.zeros((N, D), jnp.float32)
    deg_init = jnp.stack([jnp.ones((N, DEGW), jnp.float32),
                          jnp.zeros((N, DEGW), jnp.float32)])
    b1r = b1.reshape(1, D)
    b2r = b2.reshape(1, D)

    deg2 = _sc_degree(dst, deg_init)
    g1, dinv = _tc1(x, W1, deg2)
    agg1 = _sc_scatter(g1, src, dst, zeros)
    g2 = _tc2(agg1, dinv, b1r, W2)
    agg2 = _sc_scatter(g2, src, dst, zeros)
    out = _tc3(agg2, dinv, b2r)
    return (out, target)


def kernel(x, edge_index, target, W1, b1, W2, b2):
    return _run(x, edge_index, target, W1, b1, W2, b2)
